# baseline (device time: 199527 ns/iter reference)
import jax
import jax.numpy as jnp
from jax import lax
from jax.experimental import pallas as pl
from jax.experimental.pallas import tpu as pltpu

N_DEV = 16


def kernel(x, Win0, Wout0, Win1, Wout1, Win2, Wout2):
    b_sh, d_model = x.shape
    b_full = N_DEV * b_sh

    def body(x_ref, win0_ref, wout0_ref, win1_ref, wout1_ref, win2_ref,
             wout2_ref, out_ref, xfull_ref, part_ref, rsbuf_ref, myslice_ref,
             ag_send, ag_recv, rs_send, rs_recv):
        me = lax.axis_index("i")
        wins = [win0_ref, win1_ref, win2_ref]
        wouts = [wout0_ref, wout1_ref, wout2_ref]

        def row_block(ref, s):
            return ref.at[pl.ds(s * b_sh, b_sh), :]

        def ag_phase(dst_ref, final_sends):
            sends = []
            for k in range(1, N_DEV):
                tgt = lax.rem(me + k, N_DEV)
                c = pltpu.make_async_remote_copy(
                    src_ref=myslice_ref,
                    dst_ref=row_block(dst_ref, me),
                    send_sem=ag_send.at[k],
                    recv_sem=ag_recv.at[me],
                    device_id=(tgt,),
                    device_id_type=pl.DeviceIdType.MESH,
                )
                c.start()
                sends.append(c)
            dst_ref[pl.ds(me * b_sh, b_sh), :] = myslice_ref[...]
            for k in range(1, N_DEV):
                src = lax.rem(me + k, N_DEV)
                r = pltpu.make_async_remote_copy(
                    src_ref=myslice_ref,
                    dst_ref=row_block(dst_ref, src),
                    send_sem=ag_send.at[k],
                    recv_sem=ag_recv.at[src],
                    device_id=(me,),
                    device_id_type=pl.DeviceIdType.MESH,
                )
                r.wait_recv()
            final_sends.extend(sends)

        for l in range(3):
            if l == 0:
                myslice_ref[...] = x_ref[...]
            layer_sends = []
            ag_phase(xfull_ref, layer_sends)

            h = jnp.maximum(
                jnp.dot(xfull_ref[...], wins[l][...],
                        preferred_element_type=jnp.float32),
                0.0,
            )
            part_ref[...] = jnp.dot(h, wouts[l][...],
                                    preferred_element_type=jnp.float32)

            for k in range(1, N_DEV):
                tgt = lax.rem(me + k, N_DEV)
                c = pltpu.make_async_remote_copy(
                    src_ref=row_block(part_ref, tgt),
                    dst_ref=rsbuf_ref.at[me],
                    send_sem=rs_send.at[k],
                    recv_sem=rs_recv.at[me],
                    device_id=(tgt,),
                    device_id_type=pl.DeviceIdType.MESH,
                )
                c.start()
                layer_sends.append(c)
            acc = part_ref[pl.ds(me * b_sh, b_sh), :]
            for k in range(1, N_DEV):
                src = lax.rem(me + k, N_DEV)
                r = pltpu.make_async_remote_copy(
                    src_ref=myslice_ref,
                    dst_ref=rsbuf_ref.at[src],
                    send_sem=rs_send.at[k],
                    recv_sem=rs_recv.at[src],
                    device_id=(me,),
                    device_id_type=pl.DeviceIdType.MESH,
                )
                r.wait_recv()
                acc = acc + rsbuf_ref[src]
            myslice_ref[...] = acc

            for c in layer_sends:
                c.wait_send()

        final_sends = []
        ag_phase(out_ref, final_sends)
        for c in final_sends:
            c.wait_send()

    return pl.pallas_call(
        body,
        out_shape=jax.ShapeDtypeStruct((b_full, d_model), jnp.float32),
        in_specs=[pl.BlockSpec(memory_space=pltpu.VMEM)] * 7,
        out_specs=pl.BlockSpec(memory_space=pltpu.VMEM),
        scratch_shapes=[
            pltpu.VMEM((b_full, d_model), jnp.float32),
            pltpu.VMEM((b_full, d_model), jnp.float32),
            pltpu.VMEM((N_DEV, b_sh, d_model), jnp.float32),
            pltpu.VMEM((b_sh, d_model), jnp.float32),
            pltpu.SemaphoreType.DMA((N_DEV,)),
            pltpu.SemaphoreType.DMA((N_DEV,)),
            pltpu.SemaphoreType.DMA((N_DEV,)),
            pltpu.SemaphoreType.DMA((N_DEV,)),
        ],
    )(x, Win0, Wout0, Win1, Wout1, Win2, Wout2)


# device time: 186888 ns/iter; 1.0676x vs baseline; 1.0676x over previous
import jax
import jax.numpy as jnp
from jax import lax
from jax.experimental import pallas as pl
from jax.experimental.pallas import tpu as pltpu

N_DEV = 16


def kernel(x, Win0, Wout0, Win1, Wout1, Win2, Wout2):
    b_sh, d_model = x.shape
    b_full = N_DEV * b_sh

    def body(x_ref, win0_ref, wout0_ref, win1_ref, wout1_ref, win2_ref,
             wout2_ref, out_ref, xfull_ref, part_ref, rsbuf_ref, myslice_ref,
             ag_send, ag_recv, rs_send, rs_recv):
        me = lax.axis_index("i")
        wins = [win0_ref, win1_ref, win2_ref]
        wouts = [wout0_ref, wout1_ref, wout2_ref]

        def row_block(ref, s):
            return ref.at[pl.ds(s * b_sh, b_sh), :]

        def ag_phase(dst_ref, final_sends):
            sends = []
            for k in range(1, N_DEV):
                tgt = lax.rem(me + k, N_DEV)
                c = pltpu.make_async_remote_copy(
                    src_ref=myslice_ref,
                    dst_ref=row_block(dst_ref, me),
                    send_sem=ag_send.at[k],
                    recv_sem=ag_recv.at[me],
                    device_id=(tgt,),
                    device_id_type=pl.DeviceIdType.MESH,
                )
                c.start()
                sends.append(c)
            dst_ref[pl.ds(me * b_sh, b_sh), :] = myslice_ref[...]
            for k in range(1, N_DEV):
                src = lax.rem(me + k, N_DEV)
                r = pltpu.make_async_remote_copy(
                    src_ref=myslice_ref,
                    dst_ref=row_block(dst_ref, src),
                    send_sem=ag_send.at[k],
                    recv_sem=ag_recv.at[src],
                    device_id=(me,),
                    device_id_type=pl.DeviceIdType.MESH,
                )
                r.wait_recv()
            final_sends.extend(sends)

        for l in range(3):
            if l == 0:
                myslice_ref[...] = x_ref[...]
            layer_sends = []
            for k in range(1, N_DEV):
                tgt = lax.rem(me + k, N_DEV)
                c = pltpu.make_async_remote_copy(
                    src_ref=myslice_ref,
                    dst_ref=row_block(xfull_ref, me),
                    send_sem=ag_send.at[k],
                    recv_sem=ag_recv.at[me],
                    device_id=(tgt,),
                    device_id_type=pl.DeviceIdType.MESH,
                )
                c.start()
                layer_sends.append(c)
            xfull_ref[pl.ds(me * b_sh, b_sh), :] = myslice_ref[...]

            for k in range(N_DEV):
                s = lax.rem(me + k, N_DEV)
                if k > 0:
                    r = pltpu.make_async_remote_copy(
                        src_ref=myslice_ref,
                        dst_ref=row_block(xfull_ref, s),
                        send_sem=ag_send.at[k],
                        recv_sem=ag_recv.at[s],
                        device_id=(me,),
                        device_id_type=pl.DeviceIdType.MESH,
                    )
                    r.wait_recv()
                h = jnp.maximum(
                    jnp.dot(xfull_ref[pl.ds(s * b_sh, b_sh), :],
                            wins[l][...],
                            preferred_element_type=jnp.float32),
                    0.0,
                )
                part_ref[pl.ds(s * b_sh, b_sh), :] = jnp.dot(
                    h, wouts[l][...], preferred_element_type=jnp.float32)
                if k > 0:
                    c = pltpu.make_async_remote_copy(
                        src_ref=row_block(part_ref, s),
                        dst_ref=rsbuf_ref.at[me],
                        send_sem=rs_send.at[k],
                        recv_sem=rs_recv.at[me],
                        device_id=(s,),
                        device_id_type=pl.DeviceIdType.MESH,
                    )
                    c.start()
                    layer_sends.append(c)
            acc = part_ref[pl.ds(me * b_sh, b_sh), :]
            for k in range(1, N_DEV):
                src = lax.rem(me + k, N_DEV)
                r = pltpu.make_async_remote_copy(
                    src_ref=myslice_ref,
                    dst_ref=rsbuf_ref.at[src],
                    send_sem=rs_send.at[k],
                    recv_sem=rs_recv.at[src],
                    device_id=(me,),
                    device_id_type=pl.DeviceIdType.MESH,
                )
                r.wait_recv()
                acc = acc + rsbuf_ref[src]
            myslice_ref[...] = acc

            for c in layer_sends:
                c.wait_send()

        final_sends = []
        ag_phase(out_ref, final_sends)
        for c in final_sends:
            c.wait_send()

    return pl.pallas_call(
        body,
        out_shape=jax.ShapeDtypeStruct((b_full, d_model), jnp.float32),
        in_specs=[pl.BlockSpec(memory_space=pltpu.VMEM)] * 7,
        out_specs=pl.BlockSpec(memory_space=pltpu.VMEM),
        scratch_shapes=[
            pltpu.VMEM((b_full, d_model), jnp.float32),
            pltpu.VMEM((b_full, d_model), jnp.float32),
            pltpu.VMEM((N_DEV, b_sh, d_model), jnp.float32),
            pltpu.VMEM((b_sh, d_model), jnp.float32),
            pltpu.SemaphoreType.DMA((N_DEV,)),
            pltpu.SemaphoreType.DMA((N_DEV,)),
            pltpu.SemaphoreType.DMA((N_DEV,)),
            pltpu.SemaphoreType.DMA((N_DEV,)),
        ],
    )(x, Win0, Wout0, Win1, Wout1, Win2, Wout2)
